# baseline (speedup 1.0000x reference)
import jax
import jax.numpy as jnp
from jax.experimental import pallas as pl
from jax.experimental.pallas import tpu as pltpu

DIM = 1024
HEADS = 16
BUCKET = 64
S = 2048
HD = DIM // HEADS
QBLK = 256
KBLK = 256
NQB = S // QBLK
NKB = S // KBLK


def _proj_hash_kernel(xq_ref, xk_ref, xv_ref, wq_ref, bq_ref, wk_ref, bk_ref,
                      wv_ref, bv_ref, lsh_ref,
                      Q_ref, K_ref, V_ref, qh_ref, kh_ref, vsum_ref):
    i = pl.program_id(0)
    lsh = lsh_ref[...]
    q = jnp.dot(xq_ref[...], wq_ref[...],
                preferred_element_type=jnp.float32) + bq_ref[...]
    Q_ref[...] = q.astype(jnp.bfloat16)
    qh_ref[...] = jnp.argmax(jnp.dot(q, lsh, preferred_element_type=jnp.float32),
                             axis=-1).astype(jnp.int32).reshape(1, QBLK)
    k = jnp.dot(xk_ref[...], wk_ref[...],
                preferred_element_type=jnp.float32) + bk_ref[...]
    K_ref[...] = k.astype(jnp.bfloat16)
    kh_ref[...] = jnp.argmax(jnp.dot(k, lsh, preferred_element_type=jnp.float32),
                             axis=-1).astype(jnp.int32).reshape(1, QBLK)
    v = jnp.dot(xv_ref[...], wv_ref[...],
                preferred_element_type=jnp.float32) + bv_ref[...]
    V_ref[...] = v.astype(jnp.bfloat16)

    @pl.when(i == 0)
    def _():
        vsum_ref[...] = jnp.zeros((1, DIM), jnp.float32)

    vsum_ref[...] += jnp.sum(v, axis=0, keepdims=True)


def _cumsum_rows(x, n):
    shift = 1
    while shift < n:
        x = x + jnp.concatenate(
            [jnp.zeros((shift, x.shape[1]), x.dtype), x[:n - shift]], axis=0)
        shift *= 2
    return x


def _cumsum_lanes(x, n):
    shift = 1
    while shift < n:
        x = x + jnp.concatenate(
            [jnp.zeros((x.shape[0], shift), x.dtype), x[:, :n - shift]], axis=1)
        shift *= 2
    return x


def _sort_meta_kernel(qh_ref, kh_ref,
                      rankq_ref, rankk_ref, qhs_ref, khs_ref):
    def rank_of(h):
        hb = jax.lax.broadcast_in_dim(h, (S, BUCKET), (0,))
        bid = jax.lax.broadcasted_iota(jnp.int32, (S, BUCKET), 1)
        oh = (hb == bid).astype(jnp.float32)
        incl = _cumsum_rows(oh, S)
        counts = incl[S - 1:S, :]
        incl_cs = _cumsum_lanes(counts, BUCKET)
        offsets = incl_cs - counts
        rank = jnp.sum(oh * (incl - 1.0 + offsets), axis=1)
        return rank, incl_cs

    rq, q_incl_cs = rank_of(qh_ref[0, :])
    rk, k_incl_cs = rank_of(kh_ref[0, :])
    rankq_ref[...] = rq.astype(jnp.int32).reshape(1, S)
    rankk_ref[...] = rk.astype(jnp.int32).reshape(1, S)

    idx_col = jax.lax.broadcasted_iota(jnp.int32, (S, BUCKET), 0)
    qhs = jnp.sum((idx_col >= jax.lax.broadcast_in_dim(
        q_incl_cs[0].astype(jnp.int32), (S, BUCKET), (1,))).astype(jnp.int32),
        axis=1)
    qhs_ref[...] = qhs.reshape(1, S)
    khs = jnp.sum((idx_col >= jax.lax.broadcast_in_dim(
        k_incl_cs[0].astype(jnp.int32), (S, BUCKET), (1,))).astype(jnp.int32),
        axis=1)
    khs_ref[...] = khs.reshape(1, S)


def _gather_kernel(rankq_ref, rankk_ref, Q_ref, K_ref, V_ref,
                   Qs_ref, Ks_ref, Vs_ref):
    i = pl.program_id(0)
    rows = jax.lax.broadcasted_iota(jnp.int32, (QBLK, S), 0) + i * QBLK
    m2q = (rows == jax.lax.broadcast_in_dim(
        rankq_ref[0, :], (QBLK, S), (1,))).astype(jnp.bfloat16)
    m2k = (rows == jax.lax.broadcast_in_dim(
        rankk_ref[0, :], (QBLK, S), (1,))).astype(jnp.bfloat16)
    Qs_ref[...] = jnp.dot(m2q, Q_ref[...],
                          preferred_element_type=jnp.float32).astype(jnp.bfloat16)
    Ks_ref[...] = jnp.dot(m2k, K_ref[...],
                          preferred_element_type=jnp.float32).astype(jnp.bfloat16)
    Vs_ref[...] = jnp.dot(m2k, V_ref[...],
                          preferred_element_type=jnp.float32).astype(jnp.bfloat16)


def _attn_kernel(qhs_v_ref, qhs_s_ref, khs_s_ref, khs_v_ref,
                 Qs_ref, Ks_ref, Vs_ref, wo_ref, bo_ref, vsum_ref,
                 out_ref, acc_ref, m_ref, l_ref):
    qi = pl.program_id(0)
    lo = qhs_s_ref[0, qi * QBLK]
    hi = qhs_s_ref[0, qi * QBLK + QBLK - 1]

    acc_ref[...] = jnp.zeros((QBLK, DIM), jnp.float32)
    m_ref[...] = jnp.full((QBLK, 128), -1e9, jnp.float32)
    l_ref[...] = jnp.zeros((QBLK, 128), jnp.float32)

    qh_vec = qhs_v_ref[0, :]
    qhb = jax.lax.broadcast_in_dim(qh_vec, (QBLK, KBLK), (0,))

    for kj in range(NKB):
        klo = khs_s_ref[0, kj * KBLK]
        khi = khs_s_ref[0, kj * KBLK + KBLK - 1]
        active = jnp.logical_and(khi >= lo, klo <= hi)

        @pl.when(active)
        def _(kj=kj):
            ks = kj * KBLK
            kh_chunk = khs_v_ref[0, ks:ks + KBLK]
            mask = qhb == jax.lax.broadcast_in_dim(kh_chunk, (QBLK, KBLK), (1,))
            k_chunk = Ks_ref[ks:ks + KBLK, :]
            v_chunk = Vs_ref[ks:ks + KBLK, :]
            for h in range(HEADS):
                sl = slice(h * HD, (h + 1) * HD)
                s = jax.lax.dot_general(
                    Qs_ref[:, sl], k_chunk[:, sl], (((1,), (1,)), ((), ())),
                    preferred_element_type=jnp.float32) * 0.125
                s = jnp.where(mask, s, -1e9)
                m_prev = m_ref[:, h:h + 1]
                m_new = jnp.maximum(m_prev, jnp.max(s, axis=-1, keepdims=True))
                corr = jnp.exp(m_prev - m_new)
                e = jnp.where(mask, jnp.exp(s - m_new), 0.0)
                l_ref[:, h:h + 1] = l_ref[:, h:h + 1] * corr + \
                    jnp.sum(e, axis=-1, keepdims=True)
                acc_ref[:, sl] = acc_ref[:, sl] * corr + jnp.dot(
                    e.astype(jnp.bfloat16), v_chunk[:, sl],
                    preferred_element_type=jnp.float32)
                m_ref[:, h:h + 1] = m_new

    vs = vsum_ref[...]
    for h in range(HEADS):
        sl = slice(h * HD, (h + 1) * HD)
        l = l_ref[:, h:h + 1]
        has = l > 0.0
        row = acc_ref[:, sl] / jnp.where(has, l, 1.0)
        meanv = vs[0, sl] * (1.0 / S)
        acc_ref[:, sl] = jnp.where(
            jax.lax.broadcast_in_dim(has[:, 0], (QBLK, HD), (0,)),
            row, jax.lax.broadcast_in_dim(meanv, (QBLK, HD), (1,)))
    out_ref[...] = jnp.dot(acc_ref[...], wo_ref[...],
                           preferred_element_type=jnp.float32) + bo_ref[...]


def kernel(query, key, value, Wq, bq, Wk, bk, Wv, bv, Wo, bo, lsh_proj):
    xq, xk, xv = query[0], key[0], value[0]
    bq2, bk2, bv2, bo2 = (b.reshape(1, DIM) for b in (bq, bk, bv, bo))

    full = lambda shape: pl.BlockSpec(shape, lambda i: (0, 0))
    rowblk = pl.BlockSpec((QBLK, DIM), lambda i: (i, 0))
    hashblk = pl.BlockSpec((1, QBLK), lambda i: (0, i))

    Q, K, V, qh, kh, vsum = pl.pallas_call(
        _proj_hash_kernel,
        grid=(NQB,),
        in_specs=[rowblk, rowblk, rowblk,
                  full((DIM, DIM)), full((1, DIM)),
                  full((DIM, DIM)), full((1, DIM)),
                  full((DIM, DIM)), full((1, DIM)),
                  full((DIM, BUCKET))],
        out_specs=[rowblk, rowblk, rowblk, hashblk, hashblk, full((1, DIM))],
        out_shape=[jax.ShapeDtypeStruct((S, DIM), jnp.bfloat16),
                   jax.ShapeDtypeStruct((S, DIM), jnp.bfloat16),
                   jax.ShapeDtypeStruct((S, DIM), jnp.bfloat16),
                   jax.ShapeDtypeStruct((1, S), jnp.int32),
                   jax.ShapeDtypeStruct((1, S), jnp.int32),
                   jax.ShapeDtypeStruct((1, DIM), jnp.float32)],
    )(xq, xk, xv, Wq, bq2, Wk, bk2, Wv, bv2, lsh_proj)

    rankq, rankk, qhs, khs = pl.pallas_call(
        _sort_meta_kernel,
        grid=(1,),
        in_specs=[full((1, S)), full((1, S))],
        out_specs=[full((1, S)), full((1, S)), full((1, S)), full((1, S))],
        out_shape=[jax.ShapeDtypeStruct((1, S), jnp.int32)] * 4,
    )(qh, kh)

    Qs, Ks, Vs = pl.pallas_call(
        _gather_kernel,
        grid=(NQB,),
        in_specs=[full((1, S)), full((1, S)),
                  full((S, DIM)), full((S, DIM)), full((S, DIM))],
        out_specs=[rowblk, rowblk, rowblk],
        out_shape=[jax.ShapeDtypeStruct((S, DIM), jnp.bfloat16)] * 3,
    )(rankq, rankk, Q, K, V)

    out = pl.pallas_call(
        _attn_kernel,
        grid=(NQB,),
        in_specs=[hashblk,
                  pl.BlockSpec(memory_space=pltpu.SMEM),
                  pl.BlockSpec(memory_space=pltpu.SMEM),
                  full((1, S)),
                  rowblk, full((S, DIM)), full((S, DIM)),
                  full((DIM, DIM)), full((1, DIM)), full((1, DIM))],
        out_specs=rowblk,
        out_shape=jax.ShapeDtypeStruct((S, DIM), jnp.float32),
        scratch_shapes=[pltpu.VMEM((QBLK, DIM), jnp.float32),
                        pltpu.VMEM((QBLK, 128), jnp.float32),
                        pltpu.VMEM((QBLK, 128), jnp.float32)],
    )(qhs, qhs, khs, khs, Qs, Ks, Vs, Wo, bo2, vsum)

    return out.reshape(1, S, DIM)


# speedup vs baseline: 3.1761x; 3.1761x over previous
import jax
import jax.numpy as jnp
from jax.experimental import pallas as pl

DIM = 1024
HEADS = 16
BUCKET = 64
S = 2048
HD = DIM // HEADS
QBLK = 256
NQB = S // QBLK


def _proj_hash_kernel(xq_ref, xk_ref, xv_ref, wq_ref, bq_ref, wk_ref, bk_ref,
                      wv_ref, bv_ref, lsh_ref,
                      Q_ref, K_ref, V_ref, qh_ref, kh_ref, vsum_ref):
    lsh = lsh_ref[...]
    q = jnp.dot(xq_ref[...], wq_ref[...],
                preferred_element_type=jnp.float32) + bq_ref[...]
    Q_ref[...] = q
    qh_ref[...] = jnp.argmax(jnp.dot(q, lsh, preferred_element_type=jnp.float32),
                             axis=-1).astype(jnp.int32).reshape(1, QBLK)
    k = jnp.dot(xk_ref[...], wk_ref[...],
                preferred_element_type=jnp.float32) + bk_ref[...]
    K_ref[...] = k
    kh_ref[...] = jnp.argmax(jnp.dot(k, lsh, preferred_element_type=jnp.float32),
                             axis=-1).astype(jnp.int32).reshape(1, QBLK)
    v = jnp.dot(xv_ref[...], wv_ref[...],
                preferred_element_type=jnp.float32) + bv_ref[...]
    V_ref[...] = v

    i = pl.program_id(0)

    @pl.when(i == 0)
    def _():
        vsum_ref[...] = jnp.zeros((1, DIM), jnp.float32)

    vsum_ref[...] += jnp.sum(v, axis=0, keepdims=True)


def _attn_kernel(qh_ref, kh_ref, Q_ref, K_ref, V_ref, vsum_ref, o_ref):
    qi = pl.program_id(0)
    qh = qh_ref[0, pl.ds(qi * QBLK, QBLK)]
    kh = kh_ref[0, :]
    qhb = jax.lax.broadcast_in_dim(qh, (QBLK, S), (0,))
    khb = jax.lax.broadcast_in_dim(kh, (QBLK, S), (1,))
    mask = qhb == khb
    maskadd = jnp.where(mask, 0.0, -1e9)
    has = jnp.max(mask.astype(jnp.float32), axis=-1, keepdims=True) > 0.0
    vs = vsum_ref[...]
    qblk = Q_ref[...] * 0.125
    for h in range(HEADS):
        sl = slice(h * HD, (h + 1) * HD)
        s = jax.lax.dot_general(qblk[:, sl], K_ref[:, sl],
                                (((1,), (1,)), ((), ())),
                                preferred_element_type=jnp.float32) + maskadd
        m = jnp.max(s, axis=-1, keepdims=True)
        e = jnp.exp(s - m)
        o = jnp.dot(e, V_ref[:, sl], preferred_element_type=jnp.float32)
        o = o / jnp.sum(e, axis=-1, keepdims=True)
        o_ref[:, sl] = jnp.where(
            jax.lax.broadcast_in_dim(has[:, 0], (QBLK, HD), (0,)),
            o, jax.lax.broadcast_in_dim(vs[0, sl] * (1.0 / S), (QBLK, HD), (1,)))


def _rank_kernel(qh_ref, rank_ref):
    h = qh_ref[0, :]
    hb = jax.lax.broadcast_in_dim(h, (S, BUCKET), (0,))
    bid = jax.lax.broadcasted_iota(jnp.int32, (S, BUCKET), 1)
    oh = (hb == bid).astype(jnp.float32)
    incl = oh
    shift = 1
    while shift < S:
        incl = incl + jnp.concatenate(
            [jnp.zeros((shift, BUCKET), jnp.float32), incl[:S - shift]], axis=0)
        shift *= 2
    counts = incl[S - 1:S, :]
    cs = counts
    shift = 1
    while shift < BUCKET:
        cs = cs + jnp.concatenate(
            [jnp.zeros((1, shift), jnp.float32), cs[:, :BUCKET - shift]], axis=1)
        shift *= 2
    offsets = cs - counts
    rank_f = jnp.sum(oh * (incl - 1.0 + offsets), axis=1)
    rank_ref[...] = rank_f.astype(jnp.int32).reshape(1, S)


def _gather_proj_kernel(rank_ref, attn_ref, wo_ref, bo_ref, out_ref):
    i = pl.program_id(0)
    rows = jax.lax.broadcasted_iota(jnp.int32, (QBLK, S), 0) + i * QBLK
    rk = jax.lax.broadcast_in_dim(rank_ref[0, :], (QBLK, S), (1,))
    m2 = (rows == rk).astype(jnp.float32)
    g = jnp.dot(m2, attn_ref[...], preferred_element_type=jnp.float32)
    out_ref[...] = jnp.dot(g, wo_ref[...], preferred_element_type=jnp.float32) + bo_ref[...]


def kernel(query, key, value, Wq, bq, Wk, bk, Wv, bv, Wo, bo, lsh_proj):
    xq, xk, xv = query[0], key[0], value[0]
    bq2, bk2, bv2, bo2 = (b.reshape(1, DIM) for b in (bq, bk, bv, bo))

    full = lambda shape: pl.BlockSpec(shape, lambda i: (0, 0))
    rowblk = pl.BlockSpec((QBLK, DIM), lambda i: (i, 0))
    hashblk = pl.BlockSpec((1, QBLK), lambda i: (0, i))

    Q, K, V, qh, kh, vsum = pl.pallas_call(
        _proj_hash_kernel,
        grid=(NQB,),
        in_specs=[rowblk, rowblk, rowblk,
                  full((DIM, DIM)), full((1, DIM)),
                  full((DIM, DIM)), full((1, DIM)),
                  full((DIM, DIM)), full((1, DIM)),
                  full((DIM, BUCKET))],
        out_specs=[rowblk, rowblk, rowblk, hashblk, hashblk, full((1, DIM))],
        out_shape=[jax.ShapeDtypeStruct((S, DIM), jnp.float32),
                   jax.ShapeDtypeStruct((S, DIM), jnp.float32),
                   jax.ShapeDtypeStruct((S, DIM), jnp.float32),
                   jax.ShapeDtypeStruct((1, S), jnp.int32),
                   jax.ShapeDtypeStruct((1, S), jnp.int32),
                   jax.ShapeDtypeStruct((1, DIM), jnp.float32)],
    )(xq, xk, xv, Wq, bq2, Wk, bk2, Wv, bv2, lsh_proj)

    attn = pl.pallas_call(
        _attn_kernel,
        grid=(NQB,),
        in_specs=[pl.BlockSpec((1, S), lambda qi: (0, 0)),
                  pl.BlockSpec((1, S), lambda qi: (0, 0)),
                  pl.BlockSpec((QBLK, DIM), lambda qi: (qi, 0)),
                  pl.BlockSpec((S, DIM), lambda qi: (0, 0)),
                  pl.BlockSpec((S, DIM), lambda qi: (0, 0)),
                  pl.BlockSpec((1, DIM), lambda qi: (0, 0))],
        out_specs=pl.BlockSpec((QBLK, DIM), lambda qi: (qi, 0)),
        out_shape=jax.ShapeDtypeStruct((S, DIM), jnp.float32),
    )(qh, kh, Q, K, V, vsum)

    rank = pl.pallas_call(
        _rank_kernel,
        grid=(1,),
        in_specs=[pl.BlockSpec((1, S), lambda i: (0, 0))],
        out_specs=pl.BlockSpec((1, S), lambda i: (0, 0)),
        out_shape=jax.ShapeDtypeStruct((1, S), jnp.int32),
    )(qh)

    out = pl.pallas_call(
        _gather_proj_kernel,
        grid=(NQB,),
        in_specs=[pl.BlockSpec((1, S), lambda i: (0, 0)),
                  pl.BlockSpec((S, DIM), lambda i: (0, 0)),
                  pl.BlockSpec((DIM, DIM), lambda i: (0, 0)),
                  pl.BlockSpec((1, DIM), lambda i: (0, 0))],
        out_specs=rowblk,
        out_shape=jax.ShapeDtypeStruct((S, DIM), jnp.float32),
    )(rank, attn, Wo, bo2)

    return out.reshape(1, S, DIM)
